# Initial kernel scaffold; baseline (speedup 1.0000x reference)
import jax
import jax.numpy as jnp
from jax.experimental import pallas as pl


def kernel(x, edge_index, edge_attr, batch, W_enc, b_enc, W_lin, b_lin, W_edge, b_edge, gamma, beta, W_pred, b_pred):
    raise NotImplementedError("write your pallas kernel here")


# speedup vs baseline: 5.3173x; 5.3173x over previous
import functools

import jax
import jax.numpy as jnp
from jax import lax
from jax.experimental import pallas as pl
from jax.experimental.pallas import tpu as pltpu
from jax.experimental.pallas import tpu_sc as plsc

N = 10000
NPAD = 10240
PAD_NODE = 10016
E = 320000
D = 128
ED = 16
G = 64
NC = 2
NS = 16
NW = NC * NS
CH = 64
NCHUNK = 158
EPT = CH * NCHUNK
EPAD = EPT * NW
NEG = -60000.0


@functools.lru_cache(maxsize=None)
def _mesh():
    return plsc.VectorSubcoreMesh(core_axis_name="c", subcore_axis_name="s",
                                  num_cores=NC, num_subcores=NS)


def _enc_body(x_ref, w_ref, b_ref, o_ref):
    o_ref[...] = jnp.maximum(
        jnp.dot(x_ref[...], w_ref[...], preferred_element_type=jnp.float32)
        + b_ref[...], 0.0)


def _enc(x, W, b2):
    return pl.pallas_call(
        _enc_body,
        out_shape=jax.ShapeDtypeStruct((N, D), jnp.float32),
    )(x, W, b2)


def _combine_body(degp_ref, dinv_ref, invdeg_ref):
    deg = degp_ref[0, :, 0:1] + degp_ref[1, :, 0:1] + 1.0
    dinv_ref[...] = lax.rsqrt(deg)
    invdeg_ref[...] = 1.0 / deg


def _combine(degp):
    return pl.pallas_call(
        _combine_body,
        out_shape=(
            jax.ShapeDtypeStruct((NPAD, 1), jnp.float32),
            jax.ShapeDtypeStruct((NPAD, 1), jnp.float32),
        ),
    )(degp)


def _pre_body(h_ref, w_ref, b_ref, dinv_ref, invdeg_ref, hs_ref, self_ref):
    h_lin = jnp.dot(h_ref[...], w_ref[...],
                    preferred_element_type=jnp.float32) + b_ref[...]
    hs_ref[0:N, :] = dinv_ref[0:N, :] * h_lin
    hs_ref[N:NPAD, :] = jnp.zeros((NPAD - N, D), jnp.float32)
    self_ref[...] = jnp.maximum(h_lin, 0.0) * invdeg_ref[0:N, :]


def _pre(h, W, b2, dinv, invdeg):
    return pl.pallas_call(
        _pre_body,
        out_shape=(
            jax.ShapeDtypeStruct((NPAD, D), jnp.float32),
            jax.ShapeDtypeStruct((N, D), jnp.float32),
        ),
    )(h, W, b2, dinv, invdeg)


_EB = EPAD // 32


def _eemb_body(ea_ref, dr_ref, w_ref, b_ref, o_ref):
    p = pl.program_id(0)
    gid = lax.broadcasted_iota(jnp.int32, (_EB, 1), 0) + p * _EB
    emb = jnp.dot(ea_ref[...], w_ref[...],
                  preferred_element_type=jnp.float32) + b_ref[...]
    emb = dr_ref[...] * emb
    o_ref[...] = jnp.where(gid < E, emb, NEG)


def _eemb(ea_pad, dr, W_e, b2):
    return pl.pallas_call(
        _eemb_body,
        grid=(32,),
        in_specs=[
            pl.BlockSpec((_EB, ED), lambda p: (p, 0)),
            pl.BlockSpec((_EB, 1), lambda p: (p, 0)),
            pl.BlockSpec((ED, D), lambda p: (0, 0)),
            pl.BlockSpec((1, D), lambda p: (0, 0)),
        ],
        out_specs=pl.BlockSpec((_EB, D), lambda p: (p, 0)),
        out_shape=jax.ShapeDtypeStruct((EPAD, D), jnp.float32),
    )(ea_pad, dr, W_e, b2)


def _post_body(p_ref, self_ref, dinv_ref, g_ref, be_ref, o_ref, *, last):
    hpre = (p_ref[0, 0:N, :] + p_ref[1, 0:N, :]) * dinv_ref[0:N, :] \
        + self_ref[...]
    mu = jnp.mean(hpre, axis=0, keepdims=True)
    var = jnp.mean((hpre - mu) * (hpre - mu), axis=0, keepdims=True)
    hn = (hpre - mu) * lax.rsqrt(var + 1e-5) * g_ref[...] + be_ref[...]
    if not last:
        hn = jnp.maximum(hn, 0.0)
    o_ref[...] = hn


def _post(partials, selfterm, dinv, g2, be2, last):
    return pl.pallas_call(
        functools.partial(_post_body, last=last),
        out_shape=jax.ShapeDtypeStruct((N, D), jnp.float32),
    )(partials, selfterm, dinv, g2, be2)


def _pool_body(h_ref, b_ref, wp_ref, bp_ref, o_ref):
    gi = lax.broadcasted_iota(jnp.int32, (G, N), 0)
    onehot = (gi == b_ref[...]).astype(jnp.float32)
    sums = jnp.dot(onehot, h_ref[...], preferred_element_type=jnp.float32)
    counts = jnp.sum(onehot, axis=1, keepdims=True)
    hg = sums / jnp.maximum(counts, 1.0)
    o_ref[...] = jnp.dot(hg, wp_ref[...],
                         preferred_element_type=jnp.float32) + bp_ref[...]


def _pool(h, batch2, W_pred, bp2):
    return pl.pallas_call(
        _pool_body,
        out_shape=jax.ShapeDtypeStruct((G, 1), jnp.float32),
    )(h, batch2, W_pred, bp2)


def _prefetch_idx(src_hbm, dst_vmem, base, sem):
    def body(j, _):
        pltpu.async_copy(src_hbm.at[pl.ds(base + j * CH, CH)],
                         dst_vmem.at[j], sem)
        return 0
    lax.fori_loop(0, NCHUNK, body, 0)


def _drain_idx(src_hbm, dst_vmem, base, sem):
    def body(j, _):
        pltpu.make_async_copy(src_hbm.at[pl.ds(base + j * CH, CH)],
                              dst_vmem.at[j], sem).wait()
        return 0
    lax.fori_loop(0, NCHUNK, body, 0)


@functools.lru_cache(maxsize=None)
def _deg_kernel_fn():
    @functools.partial(
        pl.kernel,
        out_type=jax.ShapeDtypeStruct((NC, NPAD, D), jnp.float32),
        mesh=_mesh(),
        compiler_params=pltpu.CompilerParams(needs_layout_passes=False),
        scratch_types=[
            pltpu.VMEM((NCHUNK, CH), jnp.int32),
            pltpu.VMEM((CH, D), jnp.float32),
            pltpu.VMEM_SHARED((NPAD, D), jnp.float32),
            pltpu.SemaphoreType.DMA,
        ],
    )
    def deg_kernel(row_hbm, out_hbm, idxb, onesb, hist, sem):
        c = lax.axis_index("c")
        s = lax.axis_index("s")
        wid = s * NC + c
        stripe = NPAD // NS
        ebase = wid * EPT
        _prefetch_idx(row_hbm, idxb, ebase, sem)

        def fill(val):
            def body(i, _):
                for k in range(D // 16):
                    onesb[i, pl.ds(k * 16, 16)] = jnp.full((16,), val,
                                                           jnp.float32)
                return 0
            lax.fori_loop(0, CH, body, 0)

        fill(0.0)
        base = s * stripe
        for off in range(0, stripe, CH):
            pltpu.sync_copy(onesb, hist.at[pl.ds(base + off, CH)])
        fill(1.0)
        _drain_idx(row_hbm, idxb, ebase, sem)
        plsc.subcore_barrier()

        def chunk_body(j, _):
            pltpu.sync_copy(onesb, hist.at[idxb.at[j]], add=True)
            return 0
        lax.fori_loop(0, NCHUNK, chunk_body, 0)
        plsc.subcore_barrier()
        for off in range(0, stripe, CH):
            pltpu.sync_copy(hist.at[pl.ds(base + off, CH)],
                            out_hbm.at[c, pl.ds(base + off, CH)])

    return deg_kernel


@functools.lru_cache(maxsize=None)
def _dr_kernel_fn():
    @functools.partial(
        pl.kernel,
        out_type=jax.ShapeDtypeStruct((EPAD,), jnp.float32),
        mesh=_mesh(),
        compiler_params=pltpu.CompilerParams(needs_layout_passes=False),
        scratch_types=[
            pltpu.VMEM((NPAD,), jnp.float32),
            pltpu.VMEM((EPT,), jnp.int32),
            pltpu.VMEM((EPT,), jnp.float32),
        ],
    )
    def dr_kernel(dinv_hbm, row_hbm, out_hbm, dinv_v, idxb, obuf):
        c = lax.axis_index("c")
        s = lax.axis_index("s")
        wid = s * NC + c
        ebase = wid * EPT
        pltpu.sync_copy(row_hbm.at[pl.ds(ebase, EPT)], idxb)
        pltpu.sync_copy(dinv_hbm, dinv_v)

        @plsc.parallel_loop(0, EPT // 16, unroll=4)
        def body(t):
            sl = pl.ds(t * 16, 16)
            obuf[sl] = plsc.load_gather(dinv_v, [idxb[sl]])
        pltpu.sync_copy(obuf, out_hbm.at[pl.ds(ebase, EPT)])

    return dr_kernel


@functools.lru_cache(maxsize=None)
def _edge_kernel_fn():
    @functools.partial(
        pl.kernel,
        out_type=jax.ShapeDtypeStruct((NC, NPAD, D), jnp.float32),
        mesh=_mesh(),
        compiler_params=pltpu.CompilerParams(needs_layout_passes=False),
        scratch_types=[
            pltpu.VMEM((2, CH), jnp.int32),
            pltpu.VMEM((2, CH), jnp.int32),
            pltpu.VMEM((2, CH, D), jnp.float32),
            pltpu.VMEM((2, CH, D), jnp.float32),
            pltpu.VMEM_SHARED((NPAD, D), jnp.float32),
            pltpu.SemaphoreType.DMA,
            pltpu.SemaphoreType.DMA,
            pltpu.SemaphoreType.DMA,
            pltpu.SemaphoreType.DMA,
        ],
    )
    def edge_kernel(hs_hbm, eemb_hbm, row_hbm, col_hbm, out_hbm,
                    rowb, colb, gath, emb, agg, sg0, sg1, se0, se1):
        c = lax.axis_index("c")
        s = lax.axis_index("s")
        wid = s * NC + c
        stripe = NPAD // NS
        ebase0 = wid * EPT
        sg = (sg0, sg1)
        se = (se0, se1)

        def zero_body(i, _):
            for k in range(D // 16):
                gath[0, i, pl.ds(k * 16, 16)] = jnp.zeros((16,), jnp.float32)
            return 0
        lax.fori_loop(0, CH, zero_body, 0)
        base = s * stripe
        for off in range(0, stripe, CH):
            pltpu.sync_copy(gath.at[0], agg.at[pl.ds(base + off, CH)])

        def issue(p, j):
            ebase = ebase0 + j * CH
            pltpu.sync_copy(row_hbm.at[pl.ds(ebase, CH)], rowb.at[p])
            pltpu.sync_copy(col_hbm.at[pl.ds(ebase, CH)], colb.at[p])
            pltpu.async_copy(hs_hbm.at[rowb.at[p]], gath.at[p], sg[p])
            pltpu.async_copy(eemb_hbm.at[pl.ds(ebase, CH)], emb.at[p], se[p])

        def process(p):
            pltpu.make_async_copy(hs_hbm.at[rowb.at[p]], gath.at[p],
                                  sg[p]).wait()
            pltpu.make_async_copy(eemb_hbm.at[pl.ds(0, CH)], emb.at[p],
                                  se[p]).wait()

            @plsc.parallel_loop(0, CH, unroll=4)
            def compute_body(i):
                for k in range(D // 16):
                    sl = pl.ds(k * 16, 16)
                    gath[p, i, sl] = jnp.maximum(
                        gath[p, i, sl] + emb[p, i, sl], 0.0)
            pltpu.sync_copy(gath.at[p], agg.at[colb.at[p]], add=True)

        issue(0, 0)
        plsc.subcore_barrier()

        def pair_body(jj, _):
            issue(1, 2 * jj + 1)
            process(0)

            @pl.when(jj < NCHUNK // 2 - 1)
            def _():
                issue(0, 2 * jj + 2)
            process(1)
            return 0
        lax.fori_loop(0, NCHUNK // 2, pair_body, 0)
        plsc.subcore_barrier()
        for off in range(0, stripe, CH):
            pltpu.sync_copy(agg.at[pl.ds(base + off, CH)],
                            out_hbm.at[c, pl.ds(base + off, CH)])

    return edge_kernel


def kernel(x, edge_index, edge_attr, batch, W_enc, b_enc, W_lin, b_lin,
           W_edge, b_edge, gamma, beta, W_pred, b_pred):
    row = edge_index[0].astype(jnp.int32)
    col = edge_index[1].astype(jnp.int32)
    npad = EPAD - E
    row_pad = jnp.concatenate([row, jnp.full((npad,), PAD_NODE, jnp.int32)])
    col_pad = jnp.concatenate([col, jnp.zeros((npad,), jnp.int32)])
    ea_pad = jnp.concatenate(
        [edge_attr.astype(jnp.float32), jnp.zeros((npad, ED), jnp.float32)])
    batch2 = batch.astype(jnp.int32).reshape(1, N)

    h = _enc(x.astype(jnp.float32), W_enc, b_enc.reshape(1, D))
    degp = _deg_kernel_fn()(row_pad)
    dinv, invdeg = _combine(degp)
    dr = _dr_kernel_fn()(dinv.reshape(NPAD), row_pad).reshape(EPAD, 1)

    eembs = [_eemb(ea_pad, dr, W_edge[layer], b_edge[layer].reshape(1, D))
             for layer in range(3)]

    for layer in range(3):
        hs_pad, selfterm = _pre(h, W_lin[layer], b_lin[layer].reshape(1, D),
                                dinv, invdeg)
        partials = _edge_kernel_fn()(hs_pad, eembs[layer], row_pad, col_pad)
        h = _post(partials, selfterm, dinv, gamma[layer].reshape(1, D),
                  beta[layer].reshape(1, D), last=(layer == 2))

    return _pool(h, batch2, W_pred, b_pred.reshape(1, 1))
